# baseline (speedup 1.0000x reference)
import functools

import jax
import jax.numpy as jnp
from jax import lax
from jax.experimental import pallas as pl
from jax.experimental.pallas import tpu as pltpu
from jax.experimental.pallas import tpu_sc as plsc

BATCH = 16384
D = 64
NC = 2
NS = 16
L = 16
NW = NC * NS
BPW = BATCH // NW
CHUNK = 128
NCHUNK = BPW // CHUNK
GPW = BPW // L

_MESH = plsc.VectorSubcoreMesh(
    core_axis_name="c", subcore_axis_name="s", num_cores=NC, num_subcores=NS
)
_PARAMS = pltpu.CompilerParams(
    needs_layout_passes=False, use_tc_tiling_on_sc=False
)


def _user_body(u_hbm, ue_hbm, rows_hbm, idx_u, rows_u, sem):
    wid = lax.axis_index("s") * NC + lax.axis_index("c")
    base = wid * BPW
    pltpu.sync_copy(u_hbm.at[pl.ds(base, BPW)], idx_u)
    copies = []
    for c in range(NCHUNK):
        sl = pl.ds(c * CHUNK, CHUNK)
        copies.append(pltpu.async_copy(ue_hbm.at[idx_u.at[sl]], rows_u.at[sl], sem))
    for cp in copies:
        cp.wait()
    pltpu.sync_copy(rows_u, rows_hbm.at[pl.ds(base, BPW), :])


@functools.partial(
    pl.kernel,
    out_type=jax.ShapeDtypeStruct((BATCH, D), jnp.float32),
    mesh=_MESH,
    scratch_types=[
        pltpu.VMEM((BPW,), jnp.int32),
        pltpu.VMEM((BPW, D), jnp.float32),
        pltpu.SemaphoreType.DMA,
    ],
    compiler_params=_PARAMS,
)
def _user_kernel(*args):
    _user_body(*args)


def _diff_body(i_hbm, j_hbm, ie_hbm, diff_hbm,
               idx_i, idx_j, rows_i, rows_j, diff_v, sem):
    wid = lax.axis_index("s") * NC + lax.axis_index("c")
    base = wid * BPW

    pltpu.sync_copy(i_hbm.at[pl.ds(base, BPW)], idx_i)
    pltpu.sync_copy(j_hbm.at[pl.ds(base, BPW)], idx_j)

    copies = []
    for c in range(NCHUNK):
        sl = pl.ds(c * CHUNK, CHUNK)
        copies.append(pltpu.async_copy(ie_hbm.at[idx_i.at[sl]], rows_i.at[sl], sem))
        copies.append(pltpu.async_copy(ie_hbm.at[idx_j.at[sl]], rows_j.at[sl], sem))
    for cp in copies:
        cp.wait()

    def row_body(r, carry):
        for c in range(D // L):
            sl = pl.ds(c * L, L)
            diff_v[r, sl] = rows_i[r, sl] - rows_j[r, sl]
        return carry

    lax.fori_loop(0, BPW, row_body, 0)
    pltpu.sync_copy(diff_v, diff_hbm.at[pl.ds(base, BPW), :])


@functools.partial(
    pl.kernel,
    out_type=jax.ShapeDtypeStruct((BATCH, D), jnp.float32),
    mesh=_MESH,
    scratch_types=[
        pltpu.VMEM((BPW,), jnp.int32),
        pltpu.VMEM((BPW,), jnp.int32),
        pltpu.VMEM((BPW, D), jnp.float32),
        pltpu.VMEM((BPW, D), jnp.float32),
        pltpu.VMEM((BPW, D), jnp.float32),
        pltpu.SemaphoreType.DMA,
    ],
    compiler_params=_PARAMS,
)
def _diff_kernel(*args):
    _diff_body(*args)


def _dot_body(rows_hbm, diff_hbm, out_hbm, ru_v, diff_v, out_v, sem):
    wid = lax.axis_index("s") * NC + lax.axis_index("c")
    base = wid * BPW

    cp_u = pltpu.async_copy(rows_hbm.at[pl.ds(base, BPW), :], ru_v, sem)
    cp_d = pltpu.async_copy(diff_hbm.at[pl.ds(base, BPW), :], diff_v, sem)
    cp_u.wait()
    cp_d.wait()

    lanes = lax.iota(jnp.int32, L)

    def group_body(g, carry):
        rows_in = g * L + lanes
        acc = jnp.zeros((L,), jnp.float32)
        for d in range(D):
            col = (lanes + d) & (D - 1)
            ue = plsc.load_gather(ru_v, [rows_in, col])
            dv = plsc.load_gather(diff_v, [rows_in, col])
            acc = acc + ue * dv
        out_v[pl.ds(g * L, L)] = acc
        return carry

    lax.fori_loop(0, GPW, group_body, 0)
    pltpu.sync_copy(out_v, out_hbm.at[pl.ds(base, BPW)])


@functools.partial(
    pl.kernel,
    out_type=jax.ShapeDtypeStruct((BATCH,), jnp.float32),
    mesh=_MESH,
    scratch_types=[
        pltpu.VMEM((BPW, D), jnp.float32),
        pltpu.VMEM((BPW, D), jnp.float32),
        pltpu.VMEM((BPW,), jnp.float32),
        pltpu.SemaphoreType.DMA,
    ],
    compiler_params=_PARAMS,
)
def _dot_kernel(*args):
    _dot_body(*args)


def kernel(u, i, j, user_emb, item_emb):
    u_rows = _user_kernel(u.astype(jnp.int32), user_emb)
    diff = _diff_kernel(i.astype(jnp.int32), j.astype(jnp.int32), item_emb)
    return _dot_kernel(u_rows, diff)


# speedup vs baseline: 1.0603x; 1.0603x over previous
import functools

import jax
import jax.numpy as jnp
from jax import lax
from jax.experimental import pallas as pl
from jax.experimental.pallas import tpu as pltpu
from jax.experimental.pallas import tpu_sc as plsc

BATCH = 16384
D = 64
PAIR = 2 * D
NC = 2
NS = 16
L = 16
NW = NC * NS
BPW = BATCH // NW
CHUNK = 128
NCHUNK = BPW // CHUNK
GPC = CHUNK // L


def _bpr_body(u_hbm, i_hbm, j_hbm, ue_hbm, ie_hbm, out_hbm,
              idx_u, idx_i, idx_j, rows_u, rows_i, rows_j, out_v, sem0, sem1):
    wid = lax.axis_index("s") * NC + lax.axis_index("c")
    base = wid * BPW

    pltpu.sync_copy(u_hbm.at[pl.ds(base, BPW)], idx_u)
    pltpu.sync_copy(i_hbm.at[pl.ds(base, BPW)], idx_i)
    pltpu.sync_copy(j_hbm.at[pl.ds(base, BPW)], idx_j)

    sems = (sem0, sem1)
    lanes = lax.iota(jnp.int32, L)

    def fire_chunk(c):
        buf = c % 2
        sl = pl.ds(c * CHUNK, CHUNK)
        return (
            pltpu.async_copy(ue_hbm.at[idx_u.at[sl]], rows_u.at[buf], sems[buf]),
            pltpu.async_copy(ie_hbm.at[idx_i.at[sl]], rows_i.at[buf], sems[buf]),
            pltpu.async_copy(ie_hbm.at[idx_j.at[sl]], rows_j.at[buf], sems[buf]),
        )

    def compute_chunk(c):
        buf = c % 2
        ru, ri, rj = rows_u.at[buf], rows_i.at[buf], rows_j.at[buf]

        def group_body(g, carry):
            o = c * CHUNK + g * L
            rows_in = g * L + lanes
            acc = jnp.zeros((L,), jnp.float32)
            for d in range(D):
                col = (lanes + d) & (D - 1)
                ue = plsc.load_gather(ru, [rows_in, col])
                ie = plsc.load_gather(ri, [rows_in, col])
                je = plsc.load_gather(rj, [rows_in, col])
                acc = acc + ue * (ie - je)
            out_v[pl.ds(o, L)] = acc
            return carry

        lax.fori_loop(0, GPC, group_body, 0)

    copies = fire_chunk(0)
    for c in range(NCHUNK):
        if c + 1 < NCHUNK:
            next_copies = fire_chunk(c + 1)
        for cp in copies:
            cp.wait()
        compute_chunk(c)
        if c + 1 < NCHUNK:
            copies = next_copies

    pltpu.sync_copy(out_v, out_hbm.at[pl.ds(base, BPW)])


@functools.partial(
    pl.kernel,
    out_type=jax.ShapeDtypeStruct((BATCH,), jnp.float32),
    mesh=plsc.VectorSubcoreMesh(
        core_axis_name="c", subcore_axis_name="s", num_cores=NC, num_subcores=NS
    ),
    scratch_types=[
        pltpu.VMEM((BPW,), jnp.int32),
        pltpu.VMEM((BPW,), jnp.int32),
        pltpu.VMEM((BPW,), jnp.int32),
        pltpu.VMEM((2, CHUNK, PAIR), jnp.float32),
        pltpu.VMEM((2, CHUNK, PAIR), jnp.float32),
        pltpu.VMEM((2, CHUNK, PAIR), jnp.float32),
        pltpu.VMEM((BPW,), jnp.float32),
        pltpu.SemaphoreType.DMA,
        pltpu.SemaphoreType.DMA,
    ],
    compiler_params=pltpu.CompilerParams(
        needs_layout_passes=False, use_tc_tiling_on_sc=True
    ),
)
def _bpr_kernel(*args):
    _bpr_body(*args)


def kernel(u, i, j, user_emb, item_emb):
    ue_p = jnp.pad(user_emb, ((0, 0), (0, D)))
    ie_p = jnp.pad(item_emb, ((0, 0), (0, D)))
    return _bpr_kernel(
        u.astype(jnp.int32), i.astype(jnp.int32), j.astype(jnp.int32),
        ue_p, ie_p,
    )
